# baseline (speedup 1.0000x reference)
import functools

import jax
import jax.numpy as jnp
from jax import lax
from jax.experimental import pallas as pl
from jax.experimental.pallas import tpu as pltpu
from jax.experimental.pallas import tpu_sc as plsc

N_ENT = 10000
N_REL = 480
H = 128
E = 320000
NC = 2
NS = 16
NW = NC * NS
EPW = E // NW
CH = 128
NCH = EPW // CH
TAIL = EPW - NCH * CH
SLOPE = (1.0 / 8.0 + 1.0 / 3.0) / 2.0

_Z_ENT = 632
N_ENT_PAD = _Z_ENT * NS
_Z_REL = 32
N_REL_PAD = _Z_REL * NS

NQ = 4
QWID = N_REL // NQ
QW = N_ENT_PAD * H
_Z_Q = QW // NS
ZB = 16384

_mesh = plsc.VectorSubcoreMesh(core_axis_name="c", subcore_axis_name="s",
                               num_cores=NC, num_subcores=NS)
_sc_params = pltpu.CompilerParams(needs_layout_passes=False)


def _zero_vec(ref, n):
    z = jnp.zeros((16,), jnp.float32)

    def body(i, _):
        ref[pl.ds(i * 16, 16)] = z
        return 0

    lax.fori_loop(0, n // 16, body, 0)


def _zero_rows(rows, n, width):
    z = jnp.zeros((16,), jnp.float32)

    def body(i, _):
        for j in range(width // 16):
            rows[i, pl.ds(j * 16, 16)] = z
        return 0

    lax.fori_loop(0, n, body, 0)


def _zero_shared_stripe(rows, acc, sid, stripe):
    base = sid * stripe
    full, rem = stripe // CH, stripe % CH
    for k in range(full):
        pltpu.sync_copy(rows.at[pl.ds(0, CH)], acc.at[pl.ds(base + k * CH, CH)])
    if rem:
        pltpu.sync_copy(rows.at[pl.ds(0, rem)], acc.at[pl.ds(base + full * CH, rem)])


def _hist16(cnt, d):
    c, last = plsc.scan_count(d)
    plsc.addupdate_scatter(cnt, [d], c.astype(jnp.float32), mask=last)


@functools.partial(
    pl.kernel,
    out_type=(jax.ShapeDtypeStruct((NC, N_ENT_PAD, H), jnp.float32),
              jax.ShapeDtypeStruct((NC, N_REL_PAD, H), jnp.float32),
              jax.ShapeDtypeStruct((NC, NS, N_REL_PAD), jnp.float32)),
    mesh=_mesh,
    scratch_types=[
        pltpu.VMEM((CH,), jnp.int32),
        pltpu.VMEM((CH,), jnp.int32),
        pltpu.VMEM((CH,), jnp.int32),
        pltpu.VMEM((CH,), jnp.int32),
        pltpu.VMEM((CH,), jnp.int32),
        pltpu.VMEM((CH,), jnp.int32),
        pltpu.VMEM((CH,), jnp.int32),
        pltpu.VMEM((CH,), jnp.int32),
        pltpu.VMEM((CH, H), jnp.float32),
        pltpu.VMEM((CH, H), jnp.float32),
        pltpu.VMEM((N_REL_PAD,), jnp.float32),
        pltpu.VMEM((TAIL,), jnp.int32),
        pltpu.VMEM((TAIL,), jnp.int32),
        pltpu.VMEM((TAIL,), jnp.int32),
        pltpu.VMEM_SHARED((N_ENT_PAD, H), jnp.float32),
        pltpu.VMEM_SHARED((N_REL_PAD, H), jnp.float32),
        pltpu.SemaphoreType.DMA,
        pltpu.SemaphoreType.DMA,
        pltpu.SemaphoreType.DMA,
        pltpu.SemaphoreType.DMA,
        pltpu.SemaphoreType.DMA,
        pltpu.SemaphoreType.DMA,
    ],
    compiler_params=_sc_params,
)
def _edge_pass_a(h_hbm, src_hbm, et_hbm, dst_hbm,
                 dstsum_hbm, relsum_hbm, relcnt_hbm,
                 srcb0, srcb1, etl0, etl1, dstl0, dstl1, etb, dstb,
                 rows0, rows1, rel_cnt, srct, etbt, dstbt,
                 dst_acc, rel_acc, g0, g1, i0, i1, s0, s1):
    cid = lax.axis_index("c")
    sid = lax.axis_index("s")
    wid = sid * NC + cid

    _zero_rows(rows0, CH, H)
    _zero_vec(rel_cnt, N_REL_PAD)
    _zero_shared_stripe(rows0, dst_acc, sid, _Z_ENT)
    _zero_shared_stripe(rows0, rel_acc, sid, _Z_REL)
    plsc.subcore_barrier()

    def ioff(c):
        return pl.multiple_of(wid * EPW + c * CH, 8)

    def issue_idx(c, sb, eb, db, sem):
        pltpu.async_copy(src_hbm.at[pl.ds(ioff(c), CH)], sb, sem)
        pltpu.async_copy(et_hbm.at[pl.ds(ioff(c), CH)], eb, sem)
        pltpu.async_copy(dst_hbm.at[pl.ds(ioff(c), CH)], db, sem)

    def drain_idx(c, sb, eb, db, sem):
        pltpu.make_async_copy(src_hbm.at[pl.ds(ioff(c), CH)], sb, sem).wait()
        pltpu.make_async_copy(et_hbm.at[pl.ds(ioff(c), CH)], eb, sem).wait()
        pltpu.make_async_copy(dst_hbm.at[pl.ds(ioff(c), CH)], db, sem).wait()

    def stage(eb, db):
        for j in range(CH // 16):
            et_v = eb[pl.ds(j * 16, 16)]
            etb[pl.ds(j * 16, 16)] = et_v
            _hist16(rel_cnt, et_v)
            dstb[pl.ds(j * 16, 16)] = db[pl.ds(j * 16, 16)]

    def scat(rows):
        d1 = pltpu.async_copy(rows, dst_acc.at[dstb], s0, add=True)
        d2 = pltpu.async_copy(rows, rel_acc.at[etb], s1, add=True)
        d1.wait()
        d2.wait()

    issue_idx(0, srcb0, etl0, dstl0, i0)
    drain_idx(0, srcb0, etl0, dstl0, i0)
    pltpu.async_copy(h_hbm.at[srcb0], rows0, g0)
    issue_idx(1, srcb1, etl1, dstl1, i1)

    def pair(p, _):
        c0 = 2 * p
        drain_idx(c0 + 1, srcb1, etl1, dstl1, i1)
        pltpu.async_copy(h_hbm.at[srcb1], rows1, g1)
        pltpu.make_async_copy(h_hbm.at[srcb0], rows0, g0).wait()
        stage(etl0, dstl0)

        @pl.when(c0 + 2 < NCH)
        def _():
            issue_idx(c0 + 2, srcb0, etl0, dstl0, i0)

        scat(rows0)

        @pl.when(c0 + 2 < NCH)
        def _():
            drain_idx(c0 + 2, srcb0, etl0, dstl0, i0)
            pltpu.async_copy(h_hbm.at[srcb0], rows0, g0)

        pltpu.make_async_copy(h_hbm.at[srcb1], rows1, g1).wait()
        stage(etl1, dstl1)

        @pl.when(c0 + 3 < NCH)
        def _():
            issue_idx(c0 + 3, srcb1, etl1, dstl1, i1)

        scat(rows1)
        return 0

    lax.fori_loop(0, NCH // 2, pair, 0)
    tb = pl.multiple_of(wid * EPW + NCH * CH, 8)
    pltpu.sync_copy(src_hbm.at[pl.ds(tb, TAIL)], srct)
    pltpu.sync_copy(et_hbm.at[pl.ds(tb, TAIL)], etbt)
    pltpu.sync_copy(dst_hbm.at[pl.ds(tb, TAIL)], dstbt)
    pltpu.async_copy(h_hbm.at[srct], rows0.at[pl.ds(0, TAIL)], g0).wait()
    _hist16(rel_cnt, etbt[...])
    pltpu.sync_copy(rows0.at[pl.ds(0, TAIL)], dst_acc.at[dstbt], add=True)
    pltpu.sync_copy(rows0.at[pl.ds(0, TAIL)], rel_acc.at[etbt], add=True)
    plsc.subcore_barrier()

    pltpu.sync_copy(dst_acc.at[pl.ds(sid * _Z_ENT, _Z_ENT)],
                    dstsum_hbm.at[cid, pl.ds(sid * _Z_ENT, _Z_ENT)])
    pltpu.sync_copy(rel_acc.at[pl.ds(sid * _Z_REL, _Z_REL)],
                    relsum_hbm.at[cid, pl.ds(sid * _Z_REL, _Z_REL)])
    pltpu.sync_copy(rel_cnt, relcnt_hbm.at[cid, sid])


@functools.partial(
    pl.kernel,
    out_type=(jax.ShapeDtypeStruct((NQ * QW,), jnp.float32),
              jax.ShapeDtypeStruct((NQ * QW,), jnp.float32)),
    mesh=_mesh,
    scratch_types=[
        pltpu.VMEM((CH,), jnp.int32),
        pltpu.VMEM((CH,), jnp.int32),
        pltpu.VMEM((CH,), jnp.int32),
        pltpu.VMEM((CH,), jnp.int32),
        pltpu.VMEM((CH,), jnp.int32),
        pltpu.VMEM((CH,), jnp.int32),
        pltpu.VMEM((CH,), jnp.float32),
        pltpu.VMEM((CH,), jnp.float32),
        pltpu.VMEM((ZB,), jnp.float32),
        pltpu.VMEM((TAIL,), jnp.int32),
        pltpu.VMEM((TAIL,), jnp.int32),
        pltpu.VMEM((TAIL,), jnp.int32),
        pltpu.VMEM((TAIL,), jnp.float32),
        pltpu.VMEM_SHARED((QW,), jnp.float32),
        pltpu.SemaphoreType.DMA,
        pltpu.SemaphoreType.DMA,
        pltpu.SemaphoreType.DMA,
        pltpu.SemaphoreType.DMA,
    ],
    compiler_params=_sc_params,
)
def _cnt_pass(et_hbm, dst_hbm, out0_hbm, out1_hbm,
              majb0, majb1, minb0, minb1, linb0, linb1, valb0, valb1,
              zbuf, majt, mint, lint, valt, cflat, i0, i1, s0, s1):
    cid = lax.axis_index("c")
    sid = lax.axis_index("s")
    wid = sid * NC + cid

    _zero_vec(zbuf, ZB)

    def ioff(c):
        return pl.multiple_of(wid * EPW + c * CH, 8)

    def issue_idx(c, mj, mn, sem):
        pltpu.async_copy(dst_hbm.at[pl.ds(ioff(c), CH)], mj, sem)
        pltpu.async_copy(et_hbm.at[pl.ds(ioff(c), CH)], mn, sem)

    def drain_idx(c, mj, mn, sem):
        pltpu.make_async_copy(dst_hbm.at[pl.ds(ioff(c), CH)], mj, sem).wait()
        pltpu.make_async_copy(et_hbm.at[pl.ds(ioff(c), CH)], mn, sem).wait()

    for q in range(NQ):
        base = q * QWID
        zoff = sid * _Z_Q
        nfull = _Z_Q // ZB
        for k in range(nfull):
            pltpu.sync_copy(zbuf, cflat.at[pl.ds(zoff + k * ZB, ZB)])
        rem = _Z_Q - nfull * ZB
        if rem:
            pltpu.sync_copy(zbuf.at[pl.ds(0, rem)],
                            cflat.at[pl.ds(zoff + nfull * ZB, rem)])
        plsc.subcore_barrier()

        def compute(mj, mn, lb, vb):
            for j in range(CH // 16):
                m = mj[pl.ds(j * 16, 16)]
                off = mn[pl.ds(j * 16, 16)] - base
                inq = jnp.logical_and(off >= 0, off < QWID)
                lb[pl.ds(j * 16, 16)] = jnp.where(inq, m * H + off, 0)
                vb[pl.ds(j * 16, 16)] = jnp.where(inq, 1.0, 0.0)

        def swait(lb, vb, sem):
            pltpu.make_async_copy(vb, cflat.at[lb], sem).wait()

        issue_idx(0, majb0, minb0, i0)
        issue_idx(1, majb1, minb1, i1)

        def pair(p, _):
            c0 = 2 * p
            drain_idx(c0, majb0, minb0, i0)

            @pl.when(p > 0)
            def _():
                swait(linb0, valb0, s0)

            compute(majb0, minb0, linb0, valb0)
            pltpu.async_copy(valb0, cflat.at[linb0], s0, add=True)

            @pl.when(c0 + 2 < NCH)
            def _():
                issue_idx(c0 + 2, majb0, minb0, i0)

            drain_idx(c0 + 1, majb1, minb1, i1)

            @pl.when(p > 0)
            def _():
                swait(linb1, valb1, s1)

            compute(majb1, minb1, linb1, valb1)
            pltpu.async_copy(valb1, cflat.at[linb1], s1, add=True)

            @pl.when(c0 + 3 < NCH)
            def _():
                issue_idx(c0 + 3, majb1, minb1, i1)

            return 0

        lax.fori_loop(0, NCH // 2, pair, 0)
        swait(linb0, valb0, s0)
        swait(linb1, valb1, s1)
        tb = pl.multiple_of(wid * EPW + NCH * CH, 8)
        pltpu.sync_copy(dst_hbm.at[pl.ds(tb, TAIL)], majt)
        pltpu.sync_copy(et_hbm.at[pl.ds(tb, TAIL)], mint)
        off = mint[...] - base
        inq = jnp.logical_and(off >= 0, off < QWID)
        lint[...] = jnp.where(inq, majt[...] * H + off, 0)
        valt[...] = jnp.where(inq, 1.0, 0.0)
        pltpu.sync_copy(valt, cflat.at[lint], add=True)
        plsc.subcore_barrier()

        @pl.when(cid == 0)
        def _():
            pltpu.sync_copy(cflat.at[pl.ds(zoff, _Z_Q)],
                            out0_hbm.at[pl.ds(q * QW + zoff, _Z_Q)])

        @pl.when(cid == 1)
        def _():
            pltpu.sync_copy(cflat.at[pl.ds(zoff, _Z_Q)],
                            out1_hbm.at[pl.ds(q * QW + zoff, _Z_Q)])


def _l2n(x):
    n = jnp.sqrt(jnp.sum(x * x, axis=1, keepdims=True))
    return x / jnp.maximum(n, 1e-12)


def _init_body(emb_ref, out_ref):
    out_ref[...] = _l2n(emb_ref[...])


_init_tc = pl.pallas_call(
    _init_body, out_shape=jax.ShapeDtypeStruct((N_ENT, H), jnp.float32))


def _rel_body(parts, cnts, emb, prev, wih, whh, bih, bhh, out):
    rel_sum = parts[0, :N_REL] + parts[1, :N_REL]
    cnt = jnp.sum(cnts[...], axis=(0, 1))[:N_REL].reshape(N_REL, 1)
    x_mean = rel_sum / jnp.maximum(cnt, 1.0)
    x = jnp.concatenate([emb[...], x_mean], axis=1)
    gi = jnp.dot(x, wih[...].T, preferred_element_type=jnp.float32) + bih[...]
    gh = jnp.dot(prev[...], whh[...].T,
                 preferred_element_type=jnp.float32) + bhh[...]
    r = jax.nn.sigmoid(gi[:, :H] + gh[:, :H])
    z = jax.nn.sigmoid(gi[:, H:2 * H] + gh[:, H:2 * H])
    n = jnp.tanh(gi[:, 2 * H:] + r * gh[:, 2 * H:])
    h0 = (1.0 - z) * n + z * prev[...]
    out[...] = _l2n(h0)


_rel_tc = pl.pallas_call(
    _rel_body, out_shape=jax.ShapeDtypeStruct((N_REL, H), jnp.float32))


def _d2_body(c0, c1, h0b, d2, indeg):
    cq = c0[0, :N_ENT, :QWID] + c1[0, :N_ENT, :QWID]
    part = jnp.dot(cq, h0b[...], preferred_element_type=jnp.float32)
    ind = jnp.broadcast_to(jnp.sum(cq, axis=1, keepdims=True), (N_ENT, 8))

    @pl.when(pl.program_id(0) == 0)
    def _():
        d2[...] = part
        indeg[...] = ind

    @pl.when(pl.program_id(0) > 0)
    def _():
        d2[...] = d2[...] + part
        indeg[...] = indeg[...] + ind


_d2_tc = pl.pallas_call(
    _d2_body,
    grid=(NQ,),
    in_specs=[
        pl.BlockSpec((1, N_ENT_PAD, H), lambda q: (q, 0, 0)),
        pl.BlockSpec((1, N_ENT_PAD, H), lambda q: (q, 0, 0)),
        pl.BlockSpec((QWID, H), lambda q: (q, 0)),
    ],
    out_specs=[
        pl.BlockSpec((N_ENT, H), lambda q: (0, 0)),
        pl.BlockSpec((N_ENT, 8), lambda q: (0, 0)),
    ],
    out_shape=(jax.ShapeDtypeStruct((N_ENT, H), jnp.float32),
               jax.ShapeDtypeStruct((N_ENT, 8), jnp.float32)),
)


def _node_body(dsp, d2_ref, indeg_ref, ha, wn, lw, elw, tgw, tgb, out):
    dst_sum = dsp[0, :N_ENT] + dsp[1, :N_ENT]
    d2 = d2_ref[...]
    in_deg = indeg_ref[:, :1]
    h = ha[...]
    agg = jnp.dot(dst_sum + d2, wn[...], preferred_element_type=jnp.float32)
    hl = jnp.dot(h, lw[...], preferred_element_type=jnp.float32)
    he = jnp.dot(h, elw[...], preferred_element_type=jnp.float32)
    nr = agg + jnp.where(in_deg > 0, hl, he)
    cur = _l2n(jnp.where(nr >= 0, nr, SLOPE * nr))
    gate = jax.nn.sigmoid(
        jnp.dot(cur, tgw[...], preferred_element_type=jnp.float32) + tgb[...])
    out[...] = gate * cur + (1.0 - gate) * h


_node_tc = pl.pallas_call(
    _node_body, out_shape=jax.ShapeDtypeStruct((N_ENT, H), jnp.float32))


def kernel(dynamic_emb, emb_rel, W_ih, W_hh, b_ih, b_hh, W_neigh, loop_w,
           evolve_loop_w, time_gate_w, time_gate_b,
           edge_index_0, edge_type_0, edge_index_1, edge_type_1):
    bih = b_ih.reshape(1, 3 * H)
    bhh = b_hh.reshape(1, 3 * H)
    tgb = time_gate_b.reshape(1, H)

    h = _init_tc(dynamic_emb)
    h0 = emb_rel
    snaps = ((edge_index_0, edge_type_0), (edge_index_1, edge_type_1))
    for ei, et in snaps:
        src = ei[0]
        dst = ei[1]
        cnt0, cnt1 = _cnt_pass(et, dst)
        cnt0 = cnt0.reshape(NQ, N_ENT_PAD, H)
        cnt1 = cnt1.reshape(NQ, N_ENT_PAD, H)
        dstsum_p, relsum_p, relcnt_p = _edge_pass_a(h, src, et, dst)
        h0 = _rel_tc(relsum_p, relcnt_p, emb_rel, h0, W_ih, W_hh, bih, bhh)
        d2, indeg = _d2_tc(cnt0, cnt1, h0)
        h = _node_tc(dstsum_p, d2, indeg, h, W_neigh, loop_w,
                     evolve_loop_w, time_gate_w, tgb)
    return h


# speedup vs baseline: 2.0251x; 2.0251x over previous
import functools

import jax
import jax.numpy as jnp
from jax import lax
from jax.experimental import pallas as pl
from jax.experimental.pallas import tpu as pltpu
from jax.experimental.pallas import tpu_sc as plsc

N_ENT = 10000
N_REL = 480
H = 128
E = 320000
NC = 2
NS = 16
NW = NC * NS
EPW = E // NW
CH = 128
NCH = EPW // CH
TAIL = EPW - NCH * CH
SLOPE = (1.0 / 8.0 + 1.0 / 3.0) / 2.0

_Z_ENT = 632
N_ENT_PAD = _Z_ENT * NS
_Z_REL = 32
N_REL_PAD = _Z_REL * NS

NQ = 4
QWID = N_REL // NQ
QW = N_ENT_PAD * H
_Z_Q = QW // NS
ZB = 16384

_mesh = plsc.VectorSubcoreMesh(core_axis_name="c", subcore_axis_name="s",
                               num_cores=NC, num_subcores=NS)
_sc_params = pltpu.CompilerParams(needs_layout_passes=False)


def _zero_vec(ref, n):
    z = jnp.zeros((16,), jnp.float32)

    def body(i, _):
        ref[pl.ds(i * 16, 16)] = z
        return 0

    lax.fori_loop(0, n // 16, body, 0)


def _zero_rows(rows, n, width):
    z = jnp.zeros((16,), jnp.float32)

    def body(i, _):
        for j in range(width // 16):
            rows[i, pl.ds(j * 16, 16)] = z
        return 0

    lax.fori_loop(0, n, body, 0)


def _zero_shared_stripe(rows, acc, sid, stripe):
    base = sid * stripe
    full, rem = stripe // CH, stripe % CH
    for k in range(full):
        pltpu.sync_copy(rows.at[pl.ds(0, CH)], acc.at[pl.ds(base + k * CH, CH)])
    if rem:
        pltpu.sync_copy(rows.at[pl.ds(0, rem)], acc.at[pl.ds(base + full * CH, rem)])


def _hist16(cnt, d):
    c, last = plsc.scan_count(d)
    plsc.addupdate_scatter(cnt, [d], c.astype(jnp.float32), mask=last)


@functools.partial(
    pl.kernel,
    out_type=(jax.ShapeDtypeStruct((NC, N_ENT_PAD, H), jnp.float32),
              jax.ShapeDtypeStruct((NC, N_REL_PAD, H), jnp.float32),
              jax.ShapeDtypeStruct((NC, NS, N_REL_PAD), jnp.float32)),
    mesh=_mesh,
    scratch_types=[
        pltpu.VMEM((CH,), jnp.int32),
        pltpu.VMEM((CH,), jnp.int32),
        pltpu.VMEM((CH,), jnp.int32),
        pltpu.VMEM((CH,), jnp.int32),
        pltpu.VMEM((CH,), jnp.int32),
        pltpu.VMEM((CH,), jnp.int32),
        pltpu.VMEM((CH,), jnp.int32),
        pltpu.VMEM((CH,), jnp.int32),
        pltpu.VMEM((CH, H), jnp.float32),
        pltpu.VMEM((CH, H), jnp.float32),
        pltpu.VMEM((N_REL_PAD,), jnp.float32),
        pltpu.VMEM((TAIL,), jnp.int32),
        pltpu.VMEM((TAIL,), jnp.int32),
        pltpu.VMEM((TAIL,), jnp.int32),
        pltpu.VMEM_SHARED((N_ENT_PAD, H), jnp.float32),
        pltpu.VMEM_SHARED((N_REL_PAD, H), jnp.float32),
        pltpu.SemaphoreType.DMA,
        pltpu.SemaphoreType.DMA,
        pltpu.SemaphoreType.DMA,
        pltpu.SemaphoreType.DMA,
        pltpu.SemaphoreType.DMA,
        pltpu.SemaphoreType.DMA,
    ],
    compiler_params=_sc_params,
)
def _edge_pass_a(h_hbm, src_hbm, et_hbm, dst_hbm,
                 dstsum_hbm, relsum_hbm, relcnt_hbm,
                 srcb0, srcb1, etl0, etl1, dstl0, dstl1, etb, dstb,
                 rows0, rows1, rel_cnt, srct, etbt, dstbt,
                 dst_acc, rel_acc, g0, g1, i0, i1, s0, s1):
    cid = lax.axis_index("c")
    sid = lax.axis_index("s")
    wid = sid * NC + cid

    _zero_rows(rows0, CH, H)
    _zero_vec(rel_cnt, N_REL_PAD)
    _zero_shared_stripe(rows0, dst_acc, sid, _Z_ENT)
    _zero_shared_stripe(rows0, rel_acc, sid, _Z_REL)
    plsc.subcore_barrier()

    def ioff(c):
        return pl.multiple_of(wid * EPW + c * CH, 8)

    def issue_idx(c, sb, eb, db, sem):
        pltpu.async_copy(src_hbm.at[pl.ds(ioff(c), CH)], sb, sem)
        pltpu.async_copy(et_hbm.at[pl.ds(ioff(c), CH)], eb, sem)
        pltpu.async_copy(dst_hbm.at[pl.ds(ioff(c), CH)], db, sem)

    def drain_idx(c, sb, eb, db, sem):
        pltpu.make_async_copy(src_hbm.at[pl.ds(ioff(c), CH)], sb, sem).wait()
        pltpu.make_async_copy(et_hbm.at[pl.ds(ioff(c), CH)], eb, sem).wait()
        pltpu.make_async_copy(dst_hbm.at[pl.ds(ioff(c), CH)], db, sem).wait()

    def stage(eb, db):
        for j in range(CH // 16):
            et_v = eb[pl.ds(j * 16, 16)]
            etb[pl.ds(j * 16, 16)] = et_v
            _hist16(rel_cnt, et_v)
            dstb[pl.ds(j * 16, 16)] = db[pl.ds(j * 16, 16)]

    def scat(rows):
        d1 = pltpu.async_copy(rows, dst_acc.at[dstb], s0, add=True)
        d2 = pltpu.async_copy(rows, rel_acc.at[etb], s1, add=True)
        d1.wait()
        d2.wait()

    issue_idx(0, srcb0, etl0, dstl0, i0)
    drain_idx(0, srcb0, etl0, dstl0, i0)
    pltpu.async_copy(h_hbm.at[srcb0], rows0, g0)
    issue_idx(1, srcb1, etl1, dstl1, i1)

    def pair(p, _):
        c0 = 2 * p
        drain_idx(c0 + 1, srcb1, etl1, dstl1, i1)
        pltpu.async_copy(h_hbm.at[srcb1], rows1, g1)
        pltpu.make_async_copy(h_hbm.at[srcb0], rows0, g0).wait()
        stage(etl0, dstl0)

        @pl.when(c0 + 2 < NCH)
        def _():
            issue_idx(c0 + 2, srcb0, etl0, dstl0, i0)

        scat(rows0)

        @pl.when(c0 + 2 < NCH)
        def _():
            drain_idx(c0 + 2, srcb0, etl0, dstl0, i0)
            pltpu.async_copy(h_hbm.at[srcb0], rows0, g0)

        pltpu.make_async_copy(h_hbm.at[srcb1], rows1, g1).wait()
        stage(etl1, dstl1)

        @pl.when(c0 + 3 < NCH)
        def _():
            issue_idx(c0 + 3, srcb1, etl1, dstl1, i1)

        scat(rows1)
        return 0

    lax.fori_loop(0, NCH // 2, pair, 0)
    tb = pl.multiple_of(wid * EPW + NCH * CH, 8)
    pltpu.sync_copy(src_hbm.at[pl.ds(tb, TAIL)], srct)
    pltpu.sync_copy(et_hbm.at[pl.ds(tb, TAIL)], etbt)
    pltpu.sync_copy(dst_hbm.at[pl.ds(tb, TAIL)], dstbt)
    pltpu.async_copy(h_hbm.at[srct], rows0.at[pl.ds(0, TAIL)], g0).wait()
    _hist16(rel_cnt, etbt[...])
    pltpu.sync_copy(rows0.at[pl.ds(0, TAIL)], dst_acc.at[dstbt], add=True)
    pltpu.sync_copy(rows0.at[pl.ds(0, TAIL)], rel_acc.at[etbt], add=True)
    plsc.subcore_barrier()

    pltpu.sync_copy(dst_acc.at[pl.ds(sid * _Z_ENT, _Z_ENT)],
                    dstsum_hbm.at[cid, pl.ds(sid * _Z_ENT, _Z_ENT)])
    pltpu.sync_copy(rel_acc.at[pl.ds(sid * _Z_REL, _Z_REL)],
                    relsum_hbm.at[cid, pl.ds(sid * _Z_REL, _Z_REL)])
    pltpu.sync_copy(rel_cnt, relcnt_hbm.at[cid, sid])


@functools.partial(
    pl.kernel,
    out_type=(jax.ShapeDtypeStruct((NQ * QW,), jnp.float32),
              jax.ShapeDtypeStruct((NQ * QW,), jnp.float32)),
    mesh=_mesh,
    scratch_types=[
        pltpu.VMEM((CH,), jnp.int32),
        pltpu.VMEM((CH,), jnp.int32),
        pltpu.VMEM((CH,), jnp.int32),
        pltpu.VMEM((CH,), jnp.int32),
        pltpu.VMEM((CH,), jnp.int32),
        pltpu.VMEM((CH,), jnp.int32),
        pltpu.VMEM((CH,), jnp.float32),
        pltpu.VMEM((CH,), jnp.float32),
        pltpu.VMEM((ZB,), jnp.float32),
        pltpu.VMEM((TAIL,), jnp.int32),
        pltpu.VMEM((TAIL,), jnp.int32),
        pltpu.VMEM((TAIL,), jnp.int32),
        pltpu.VMEM((TAIL,), jnp.float32),
        pltpu.VMEM_SHARED((QW,), jnp.float32),
        pltpu.SemaphoreType.DMA,
        pltpu.SemaphoreType.DMA,
        pltpu.SemaphoreType.DMA,
        pltpu.SemaphoreType.DMA,
    ],
    compiler_params=_sc_params,
)
def _cnt_pass(et_hbm, dst_hbm, out0_hbm, out1_hbm,
              majb0, majb1, minb0, minb1, linb0, linb1, valb0, valb1,
              zbuf, majt, mint, lint, valt, cflat, i0, i1, s0, s1):
    cid = lax.axis_index("c")
    sid = lax.axis_index("s")
    wid = sid * NC + cid

    _zero_vec(zbuf, ZB)

    def ioff(c):
        return pl.multiple_of(wid * EPW + c * CH, 8)

    def issue_idx(c, mj, mn, sem):
        pltpu.async_copy(dst_hbm.at[pl.ds(ioff(c), CH)], mj, sem)
        pltpu.async_copy(et_hbm.at[pl.ds(ioff(c), CH)], mn, sem)

    def drain_idx(c, mj, mn, sem):
        pltpu.make_async_copy(dst_hbm.at[pl.ds(ioff(c), CH)], mj, sem).wait()
        pltpu.make_async_copy(et_hbm.at[pl.ds(ioff(c), CH)], mn, sem).wait()

    for q in range(NQ):
        base = q * QWID
        zoff = sid * _Z_Q
        nfull = _Z_Q // ZB
        for k in range(nfull):
            pltpu.sync_copy(zbuf, cflat.at[pl.ds(zoff + k * ZB, ZB)])
        rem = _Z_Q - nfull * ZB
        if rem:
            pltpu.sync_copy(zbuf.at[pl.ds(0, rem)],
                            cflat.at[pl.ds(zoff + nfull * ZB, rem)])
        plsc.subcore_barrier()

        def compute(mj, mn, lb, vb):
            for j in range(CH // 16):
                m = mj[pl.ds(j * 16, 16)]
                off = mn[pl.ds(j * 16, 16)] - base
                inq = jnp.logical_and(off >= 0, off < QWID)
                safe = jnp.where(inq, off, QWID + (j % 8))
                lb[pl.ds(j * 16, 16)] = m * H + safe
                vb[pl.ds(j * 16, 16)] = jnp.where(inq, 1.0, 0.0)

        def swait(lb, vb, sem):
            pltpu.make_async_copy(vb, cflat.at[lb], sem).wait()

        issue_idx(0, majb0, minb0, i0)
        issue_idx(1, majb1, minb1, i1)

        def pair(p, _):
            c0 = 2 * p
            drain_idx(c0, majb0, minb0, i0)

            @pl.when(p > 0)
            def _():
                swait(linb0, valb0, s0)

            compute(majb0, minb0, linb0, valb0)
            pltpu.async_copy(valb0, cflat.at[linb0], s0, add=True)

            @pl.when(c0 + 2 < NCH)
            def _():
                issue_idx(c0 + 2, majb0, minb0, i0)

            drain_idx(c0 + 1, majb1, minb1, i1)

            @pl.when(p > 0)
            def _():
                swait(linb1, valb1, s1)

            compute(majb1, minb1, linb1, valb1)
            pltpu.async_copy(valb1, cflat.at[linb1], s1, add=True)

            @pl.when(c0 + 3 < NCH)
            def _():
                issue_idx(c0 + 3, majb1, minb1, i1)

            return 0

        lax.fori_loop(0, NCH // 2, pair, 0)
        swait(linb0, valb0, s0)
        swait(linb1, valb1, s1)
        tb = pl.multiple_of(wid * EPW + NCH * CH, 8)
        pltpu.sync_copy(dst_hbm.at[pl.ds(tb, TAIL)], majt)
        pltpu.sync_copy(et_hbm.at[pl.ds(tb, TAIL)], mint)
        off = mint[...] - base
        inq = jnp.logical_and(off >= 0, off < QWID)
        lint[...] = majt[...] * H + jnp.where(inq, off, QWID)
        valt[...] = jnp.where(inq, 1.0, 0.0)
        pltpu.sync_copy(valt, cflat.at[lint], add=True)
        plsc.subcore_barrier()

        @pl.when(cid == 0)
        def _():
            pltpu.sync_copy(cflat.at[pl.ds(zoff, _Z_Q)],
                            out0_hbm.at[pl.ds(q * QW + zoff, _Z_Q)])

        @pl.when(cid == 1)
        def _():
            pltpu.sync_copy(cflat.at[pl.ds(zoff, _Z_Q)],
                            out1_hbm.at[pl.ds(q * QW + zoff, _Z_Q)])


def _l2n(x):
    n = jnp.sqrt(jnp.sum(x * x, axis=1, keepdims=True))
    return x / jnp.maximum(n, 1e-12)


def _init_body(emb_ref, out_ref):
    out_ref[...] = _l2n(emb_ref[...])


_init_tc = pl.pallas_call(
    _init_body, out_shape=jax.ShapeDtypeStruct((N_ENT, H), jnp.float32))


def _rel_body(parts, cnts, emb, prev, wih, whh, bih, bhh, out):
    rel_sum = parts[0, :N_REL] + parts[1, :N_REL]
    cnt = jnp.sum(cnts[...], axis=(0, 1))[:N_REL].reshape(N_REL, 1)
    x_mean = rel_sum / jnp.maximum(cnt, 1.0)
    x = jnp.concatenate([emb[...], x_mean], axis=1)
    gi = jnp.dot(x, wih[...].T, preferred_element_type=jnp.float32) + bih[...]
    gh = jnp.dot(prev[...], whh[...].T,
                 preferred_element_type=jnp.float32) + bhh[...]
    r = jax.nn.sigmoid(gi[:, :H] + gh[:, :H])
    z = jax.nn.sigmoid(gi[:, H:2 * H] + gh[:, H:2 * H])
    n = jnp.tanh(gi[:, 2 * H:] + r * gh[:, 2 * H:])
    h0 = (1.0 - z) * n + z * prev[...]
    out[...] = _l2n(h0)


_rel_tc = pl.pallas_call(
    _rel_body, out_shape=jax.ShapeDtypeStruct((N_REL, H), jnp.float32))


def _d2_body(c0, c1, h0b, d2, indeg):
    cq = c0[0, :N_ENT, :QWID] + c1[0, :N_ENT, :QWID]
    part = jnp.dot(cq, h0b[...], preferred_element_type=jnp.float32)
    ind = jnp.broadcast_to(jnp.sum(cq, axis=1, keepdims=True), (N_ENT, 8))

    @pl.when(pl.program_id(0) == 0)
    def _():
        d2[...] = part
        indeg[...] = ind

    @pl.when(pl.program_id(0) > 0)
    def _():
        d2[...] = d2[...] + part
        indeg[...] = indeg[...] + ind


_d2_tc = pl.pallas_call(
    _d2_body,
    grid=(NQ,),
    in_specs=[
        pl.BlockSpec((1, N_ENT_PAD, H), lambda q: (q, 0, 0)),
        pl.BlockSpec((1, N_ENT_PAD, H), lambda q: (q, 0, 0)),
        pl.BlockSpec((QWID, H), lambda q: (q, 0)),
    ],
    out_specs=[
        pl.BlockSpec((N_ENT, H), lambda q: (0, 0)),
        pl.BlockSpec((N_ENT, 8), lambda q: (0, 0)),
    ],
    out_shape=(jax.ShapeDtypeStruct((N_ENT, H), jnp.float32),
               jax.ShapeDtypeStruct((N_ENT, 8), jnp.float32)),
)


def _node_body(dsp, d2_ref, indeg_ref, ha, wn, lw, elw, tgw, tgb, out):
    dst_sum = dsp[0, :N_ENT] + dsp[1, :N_ENT]
    d2 = d2_ref[...]
    in_deg = indeg_ref[:, :1]
    h = ha[...]
    agg = jnp.dot(dst_sum + d2, wn[...], preferred_element_type=jnp.float32)
    hl = jnp.dot(h, lw[...], preferred_element_type=jnp.float32)
    he = jnp.dot(h, elw[...], preferred_element_type=jnp.float32)
    nr = agg + jnp.where(in_deg > 0, hl, he)
    cur = _l2n(jnp.where(nr >= 0, nr, SLOPE * nr))
    gate = jax.nn.sigmoid(
        jnp.dot(cur, tgw[...], preferred_element_type=jnp.float32) + tgb[...])
    out[...] = gate * cur + (1.0 - gate) * h


_node_tc = pl.pallas_call(
    _node_body, out_shape=jax.ShapeDtypeStruct((N_ENT, H), jnp.float32))


def kernel(dynamic_emb, emb_rel, W_ih, W_hh, b_ih, b_hh, W_neigh, loop_w,
           evolve_loop_w, time_gate_w, time_gate_b,
           edge_index_0, edge_type_0, edge_index_1, edge_type_1):
    bih = b_ih.reshape(1, 3 * H)
    bhh = b_hh.reshape(1, 3 * H)
    tgb = time_gate_b.reshape(1, H)

    h = _init_tc(dynamic_emb)
    h0 = emb_rel
    snaps = ((edge_index_0, edge_type_0), (edge_index_1, edge_type_1))
    for ei, et in snaps:
        src = ei[0]
        dst = ei[1]
        cnt0, cnt1 = _cnt_pass(et, dst)
        cnt0 = cnt0.reshape(NQ, N_ENT_PAD, H)
        cnt1 = cnt1.reshape(NQ, N_ENT_PAD, H)
        dstsum_p, relsum_p, relcnt_p = _edge_pass_a(h, src, et, dst)
        h0 = _rel_tc(relsum_p, relcnt_p, emb_rel, h0, W_ih, W_hh, bih, bhh)
        d2, indeg = _d2_tc(cnt0, cnt1, h0)
        h = _node_tc(dstsum_p, d2, indeg, h, W_neigh, loop_w,
                     evolve_loop_w, time_gate_w, tgb)
    return h


# speedup vs baseline: 2.5435x; 1.2560x over previous
import functools

import jax
import jax.numpy as jnp
from jax import lax
from jax.experimental import pallas as pl
from jax.experimental.pallas import tpu as pltpu
from jax.experimental.pallas import tpu_sc as plsc

N_ENT = 10000
N_REL = 480
H = 128
E = 320000
NC = 2
NS = 16
NW = NC * NS
EPW = E // NW
CH = 128
NCH = EPW // CH
TAIL = EPW - NCH * CH
SLOPE = (1.0 / 8.0 + 1.0 / 3.0) / 2.0

_Z_ENT = 632
N_ENT_PAD = _Z_ENT * NS
_Z_REL = 32
N_REL_PAD = _Z_REL * NS

_mesh = plsc.VectorSubcoreMesh(core_axis_name="c", subcore_axis_name="s",
                               num_cores=NC, num_subcores=NS)
_sc_params = pltpu.CompilerParams(needs_layout_passes=False)


def _zero_vec(ref, n):
    z = jnp.zeros((16,), jnp.float32)

    def body(i, _):
        ref[pl.ds(i * 16, 16)] = z
        return 0

    lax.fori_loop(0, n // 16, body, 0)


def _zero_rows(rows, n, width):
    z = jnp.zeros((16,), jnp.float32)

    def body(i, _):
        for j in range(width // 16):
            rows[i, pl.ds(j * 16, 16)] = z
        return 0

    lax.fori_loop(0, n, body, 0)


def _zero_shared_stripe(rows, acc, sid, stripe):
    base = sid * stripe
    full, rem = stripe // CH, stripe % CH
    for k in range(full):
        pltpu.sync_copy(rows.at[pl.ds(0, CH)], acc.at[pl.ds(base + k * CH, CH)])
    if rem:
        pltpu.sync_copy(rows.at[pl.ds(0, rem)], acc.at[pl.ds(base + full * CH, rem)])


def _hist16(cnt, d):
    c, last = plsc.scan_count(d)
    plsc.addupdate_scatter(cnt, [d], c.astype(jnp.float32), mask=last)


@functools.partial(
    pl.kernel,
    out_type=(jax.ShapeDtypeStruct((NC, N_ENT_PAD, H), jnp.float32),
              jax.ShapeDtypeStruct((NC, N_REL_PAD, H), jnp.float32),
              jax.ShapeDtypeStruct((NC, NS, N_REL_PAD), jnp.float32)),
    mesh=_mesh,
    scratch_types=[
        pltpu.VMEM((CH,), jnp.int32),
        pltpu.VMEM((CH,), jnp.int32),
        pltpu.VMEM((CH,), jnp.int32),
        pltpu.VMEM((CH,), jnp.int32),
        pltpu.VMEM((CH,), jnp.int32),
        pltpu.VMEM((CH,), jnp.int32),
        pltpu.VMEM((CH,), jnp.int32),
        pltpu.VMEM((CH,), jnp.int32),
        pltpu.VMEM((CH,), jnp.int32),
        pltpu.VMEM((CH,), jnp.int32),
        pltpu.VMEM((CH, H), jnp.float32),
        pltpu.VMEM((CH, H), jnp.float32),
        pltpu.VMEM((N_REL_PAD,), jnp.float32),
        pltpu.VMEM((TAIL,), jnp.int32),
        pltpu.VMEM((TAIL,), jnp.int32),
        pltpu.VMEM((TAIL,), jnp.int32),
        pltpu.VMEM_SHARED((N_ENT_PAD, H), jnp.float32),
        pltpu.VMEM_SHARED((N_REL_PAD, H), jnp.float32),
        pltpu.SemaphoreType.DMA,
        pltpu.SemaphoreType.DMA,
        pltpu.SemaphoreType.DMA,
        pltpu.SemaphoreType.DMA,
        pltpu.SemaphoreType.DMA,
        pltpu.SemaphoreType.DMA,
        pltpu.SemaphoreType.DMA,
        pltpu.SemaphoreType.DMA,
    ],
    compiler_params=_sc_params,
)
def _edge_pass_a(h_hbm, src_hbm, et_hbm, dst_hbm,
                 dstsum_hbm, relsum_hbm, relcnt_hbm,
                 srcb0, srcb1, etl0, etl1, dstl0, dstl1, etb0, dstb0,
                 etb1, dstb1, rows0, rows1, rel_cnt, srct, etbt, dstbt,
                 dst_acc, rel_acc, g0, g1, i0, i1, sa0, sa1, sb0, sb1):
    cid = lax.axis_index("c")
    sid = lax.axis_index("s")
    wid = sid * NC + cid

    _zero_rows(rows0, CH, H)
    _zero_vec(rel_cnt, N_REL_PAD)
    _zero_shared_stripe(rows0, dst_acc, sid, _Z_ENT)
    _zero_shared_stripe(rows0, rel_acc, sid, _Z_REL)
    plsc.subcore_barrier()

    def ioff(c):
        return pl.multiple_of(wid * EPW + c * CH, 8)

    def issue_idx(c, sb, eb, db, sem):
        pltpu.async_copy(src_hbm.at[pl.ds(ioff(c), CH)], sb, sem)
        pltpu.async_copy(et_hbm.at[pl.ds(ioff(c), CH)], eb, sem)
        pltpu.async_copy(dst_hbm.at[pl.ds(ioff(c), CH)], db, sem)

    def drain_idx(c, sb, eb, db, sem):
        pltpu.make_async_copy(src_hbm.at[pl.ds(ioff(c), CH)], sb, sem).wait()
        pltpu.make_async_copy(et_hbm.at[pl.ds(ioff(c), CH)], eb, sem).wait()
        pltpu.make_async_copy(dst_hbm.at[pl.ds(ioff(c), CH)], db, sem).wait()

    def stage(eb, db, etb, dstb):
        for j in range(CH // 16):
            et_v = eb[pl.ds(j * 16, 16)]
            etb[pl.ds(j * 16, 16)] = et_v
            _hist16(rel_cnt, et_v)
            dstb[pl.ds(j * 16, 16)] = db[pl.ds(j * 16, 16)]

    def scat_issue(rows, etb, dstb, sd, sr):
        pltpu.async_copy(rows, dst_acc.at[dstb], sd, add=True)
        pltpu.async_copy(rows, rel_acc.at[etb], sr, add=True)

    def scat_wait(rows, etb, dstb, sd, sr):
        pltpu.make_async_copy(rows, dst_acc.at[dstb], sd).wait()
        pltpu.make_async_copy(rows, rel_acc.at[etb], sr).wait()

    issue_idx(0, srcb0, etl0, dstl0, i0)
    drain_idx(0, srcb0, etl0, dstl0, i0)
    pltpu.async_copy(h_hbm.at[srcb0], rows0, g0)
    issue_idx(1, srcb1, etl1, dstl1, i1)

    def pair(p, _):
        c0 = 2 * p
        drain_idx(c0 + 1, srcb1, etl1, dstl1, i1)

        @pl.when(p > 0)
        def _():
            scat_wait(rows1, etb1, dstb1, sb0, sb1)

        pltpu.async_copy(h_hbm.at[srcb1], rows1, g1)
        pltpu.make_async_copy(h_hbm.at[srcb0], rows0, g0).wait()
        stage(etl0, dstl0, etb0, dstb0)
        scat_issue(rows0, etb0, dstb0, sa0, sa1)

        @pl.when(c0 + 2 < NCH)
        def _():
            issue_idx(c0 + 2, srcb0, etl0, dstl0, i0)
            drain_idx(c0 + 2, srcb0, etl0, dstl0, i0)
            scat_wait(rows0, etb0, dstb0, sa0, sa1)
            pltpu.async_copy(h_hbm.at[srcb0], rows0, g0)

        pltpu.make_async_copy(h_hbm.at[srcb1], rows1, g1).wait()
        stage(etl1, dstl1, etb1, dstb1)
        scat_issue(rows1, etb1, dstb1, sb0, sb1)

        @pl.when(c0 + 3 < NCH)
        def _():
            issue_idx(c0 + 3, srcb1, etl1, dstl1, i1)

        return 0

    lax.fori_loop(0, NCH // 2, pair, 0)
    scat_wait(rows0, etb0, dstb0, sa0, sa1)
    scat_wait(rows1, etb1, dstb1, sb0, sb1)
    tb = pl.multiple_of(wid * EPW + NCH * CH, 8)
    pltpu.sync_copy(src_hbm.at[pl.ds(tb, TAIL)], srct)
    pltpu.sync_copy(et_hbm.at[pl.ds(tb, TAIL)], etbt)
    pltpu.sync_copy(dst_hbm.at[pl.ds(tb, TAIL)], dstbt)
    pltpu.async_copy(h_hbm.at[srct], rows0.at[pl.ds(0, TAIL)], g0).wait()
    _hist16(rel_cnt, etbt[...])
    pltpu.sync_copy(rows0.at[pl.ds(0, TAIL)], dst_acc.at[dstbt], add=True)
    pltpu.sync_copy(rows0.at[pl.ds(0, TAIL)], rel_acc.at[etbt], add=True)
    plsc.subcore_barrier()

    pltpu.sync_copy(dst_acc.at[pl.ds(sid * _Z_ENT, _Z_ENT)],
                    dstsum_hbm.at[cid, pl.ds(sid * _Z_ENT, _Z_ENT)])
    pltpu.sync_copy(rel_acc.at[pl.ds(sid * _Z_REL, _Z_REL)],
                    relsum_hbm.at[cid, pl.ds(sid * _Z_REL, _Z_REL)])
    pltpu.sync_copy(rel_cnt, relcnt_hbm.at[cid, sid])


@functools.partial(
    pl.kernel,
    out_type=(jax.ShapeDtypeStruct((NC, N_ENT_PAD, H), jnp.float32),
              jax.ShapeDtypeStruct((NC, NS, N_ENT_PAD), jnp.float32)),
    mesh=_mesh,
    scratch_types=[
        pltpu.VMEM((CH,), jnp.int32),
        pltpu.VMEM((CH,), jnp.int32),
        pltpu.VMEM((CH,), jnp.int32),
        pltpu.VMEM((CH,), jnp.int32),
        pltpu.VMEM((CH,), jnp.int32),
        pltpu.VMEM((CH,), jnp.int32),
        pltpu.VMEM((CH, H), jnp.float32),
        pltpu.VMEM((CH, H), jnp.float32),
        pltpu.VMEM((N_ENT_PAD,), jnp.float32),
        pltpu.VMEM((TAIL,), jnp.int32),
        pltpu.VMEM((TAIL,), jnp.int32),
        pltpu.VMEM_SHARED((N_ENT_PAD, H), jnp.float32),
        pltpu.VMEM_SHARED((N_REL_PAD, H), jnp.float32),
        pltpu.SemaphoreType.DMA,
        pltpu.SemaphoreType.DMA,
        pltpu.SemaphoreType.DMA,
        pltpu.SemaphoreType.DMA,
        pltpu.SemaphoreType.DMA,
        pltpu.SemaphoreType.DMA,
    ],
    compiler_params=_sc_params,
)
def _edge_pass_c(h0_hbm, et_hbm, dst_hbm, d2_hbm, deg_hbm,
                 etl0, etl1, dstl0, dstl1, dstb0, dstb1, rows0, rows1, deg,
                 ett, dstbt, acc, h0_sh, g0, g1, i0, i1, sa, sb):
    cid = lax.axis_index("c")
    sid = lax.axis_index("s")
    wid = sid * NC + cid

    _zero_rows(rows0, CH, H)
    _zero_vec(deg, N_ENT_PAD)
    _zero_shared_stripe(rows0, acc, sid, _Z_ENT)
    @pl.when(sid < NS - 1)
    def _():
        pltpu.sync_copy(h0_hbm.at[pl.ds(sid * _Z_REL, _Z_REL)],
                        h0_sh.at[pl.ds(sid * _Z_REL, _Z_REL)])

    plsc.subcore_barrier()

    def ioff(c):
        return pl.multiple_of(wid * EPW + c * CH, 8)

    def issue_idx(c, eb, db, sem):
        pltpu.async_copy(et_hbm.at[pl.ds(ioff(c), CH)], eb, sem)
        pltpu.async_copy(dst_hbm.at[pl.ds(ioff(c), CH)], db, sem)

    def drain_idx(c, eb, db, sem):
        pltpu.make_async_copy(et_hbm.at[pl.ds(ioff(c), CH)], eb, sem).wait()
        pltpu.make_async_copy(dst_hbm.at[pl.ds(ioff(c), CH)], db, sem).wait()

    def stage(db, dstb):
        for j in range(CH // 16):
            dst_v = db[pl.ds(j * 16, 16)]
            dstb[pl.ds(j * 16, 16)] = dst_v
            _hist16(deg, dst_v)

    issue_idx(0, etl0, dstl0, i0)
    drain_idx(0, etl0, dstl0, i0)
    pltpu.async_copy(h0_sh.at[etl0], rows0, g0)
    issue_idx(1, etl1, dstl1, i1)

    def pair(p, _):
        c0 = 2 * p
        drain_idx(c0 + 1, etl1, dstl1, i1)

        @pl.when(p > 0)
        def _():
            pltpu.make_async_copy(rows1, acc.at[dstb1], sb).wait()

        pltpu.async_copy(h0_sh.at[etl1], rows1, g1)
        pltpu.make_async_copy(h0_sh.at[etl0], rows0, g0).wait()
        stage(dstl0, dstb0)
        pltpu.async_copy(rows0, acc.at[dstb0], sa, add=True)

        @pl.when(c0 + 2 < NCH)
        def _():
            issue_idx(c0 + 2, etl0, dstl0, i0)
            drain_idx(c0 + 2, etl0, dstl0, i0)
            pltpu.make_async_copy(rows0, acc.at[dstb0], sa).wait()
            pltpu.async_copy(h0_sh.at[etl0], rows0, g0)

        pltpu.make_async_copy(h0_sh.at[etl1], rows1, g1).wait()
        stage(dstl1, dstb1)
        pltpu.async_copy(rows1, acc.at[dstb1], sb, add=True)

        @pl.when(c0 + 3 < NCH)
        def _():
            issue_idx(c0 + 3, etl1, dstl1, i1)

        return 0

    lax.fori_loop(0, NCH // 2, pair, 0)
    pltpu.make_async_copy(rows0, acc.at[dstb0], sa).wait()
    pltpu.make_async_copy(rows1, acc.at[dstb1], sb).wait()
    tb = pl.multiple_of(wid * EPW + NCH * CH, 8)
    pltpu.sync_copy(et_hbm.at[pl.ds(tb, TAIL)], ett)
    pltpu.sync_copy(dst_hbm.at[pl.ds(tb, TAIL)], dstbt)
    pltpu.async_copy(h0_sh.at[ett], rows0.at[pl.ds(0, TAIL)], g0).wait()
    dst_v = dstbt[...]
    _hist16(deg, dst_v)
    pltpu.sync_copy(rows0.at[pl.ds(0, TAIL)], acc.at[dstbt], add=True)
    plsc.subcore_barrier()

    pltpu.sync_copy(acc.at[pl.ds(sid * _Z_ENT, _Z_ENT)],
                    d2_hbm.at[cid, pl.ds(sid * _Z_ENT, _Z_ENT)])
    pltpu.sync_copy(deg, deg_hbm.at[cid, sid])


def _l2n(x):
    n = jnp.sqrt(jnp.sum(x * x, axis=1, keepdims=True))
    return x / jnp.maximum(n, 1e-12)


def _init_body(emb_ref, out_ref):
    out_ref[...] = _l2n(emb_ref[...])


_init_tc = pl.pallas_call(
    _init_body, out_shape=jax.ShapeDtypeStruct((N_ENT, H), jnp.float32))


def _rel_body(parts, cnts, emb, prev, wih, whh, bih, bhh, out):
    rel_sum = parts[0, :N_REL] + parts[1, :N_REL]
    cnt = jnp.sum(cnts[...], axis=(0, 1))[:N_REL].reshape(N_REL, 1)
    x_mean = rel_sum / jnp.maximum(cnt, 1.0)
    x = jnp.concatenate([emb[...], x_mean], axis=1)
    gi = jnp.dot(x, wih[...].T, preferred_element_type=jnp.float32) + bih[...]
    gh = jnp.dot(prev[...], whh[...].T,
                 preferred_element_type=jnp.float32) + bhh[...]
    r = jax.nn.sigmoid(gi[:, :H] + gh[:, :H])
    z = jax.nn.sigmoid(gi[:, H:2 * H] + gh[:, H:2 * H])
    n = jnp.tanh(gi[:, 2 * H:] + r * gh[:, 2 * H:])
    h0 = (1.0 - z) * n + z * prev[...]
    out[...] = _l2n(h0)


_rel_tc = pl.pallas_call(
    _rel_body, out_shape=jax.ShapeDtypeStruct((N_REL, H), jnp.float32))


def _node_body(dsp, d2p, degp, ha, wn, lw, elw, tgw, tgb, out):
    dst_sum = dsp[0, :N_ENT] + dsp[1, :N_ENT]
    d2 = d2p[0, :N_ENT] + d2p[1, :N_ENT]
    in_deg = jnp.sum(degp[...], axis=(0, 1))[:N_ENT].reshape(N_ENT, 1)
    h = ha[...]
    agg = jnp.dot(dst_sum + d2, wn[...], preferred_element_type=jnp.float32)
    hl = jnp.dot(h, lw[...], preferred_element_type=jnp.float32)
    he = jnp.dot(h, elw[...], preferred_element_type=jnp.float32)
    nr = agg + jnp.where(in_deg > 0, hl, he)
    cur = _l2n(jnp.where(nr >= 0, nr, SLOPE * nr))
    gate = jax.nn.sigmoid(
        jnp.dot(cur, tgw[...], preferred_element_type=jnp.float32) + tgb[...])
    out[...] = gate * cur + (1.0 - gate) * h


_node_tc = pl.pallas_call(
    _node_body, out_shape=jax.ShapeDtypeStruct((N_ENT, H), jnp.float32))


def kernel(dynamic_emb, emb_rel, W_ih, W_hh, b_ih, b_hh, W_neigh, loop_w,
           evolve_loop_w, time_gate_w, time_gate_b,
           edge_index_0, edge_type_0, edge_index_1, edge_type_1):
    bih = b_ih.reshape(1, 3 * H)
    bhh = b_hh.reshape(1, 3 * H)
    tgb = time_gate_b.reshape(1, H)

    h = _init_tc(dynamic_emb)
    h0 = emb_rel
    snaps = ((edge_index_0, edge_type_0), (edge_index_1, edge_type_1))
    for ei, et in snaps:
        src = ei[0]
        dst = ei[1]
        dstsum_p, relsum_p, relcnt_p = _edge_pass_a(h, src, et, dst)
        h0 = _rel_tc(relsum_p, relcnt_p, emb_rel, h0, W_ih, W_hh, bih, bhh)
        d2_p, deg_p = _edge_pass_c(h0, et, dst)
        h = _node_tc(dstsum_p, d2_p, deg_p, h, W_neigh, loop_w,
                     evolve_loop_w, time_gate_w, tgb)
    return h
